# baseline (speedup 1.0000x reference)
import functools
import jax
import jax.numpy as jnp
from jax import lax
from jax.experimental import pallas as pl
from jax.experimental.pallas import tpu as pltpu
from jax.experimental.pallas import tpu_sc as plsc

N_NODES = 10000
E_EDGES = 320000
DIM = 128

NUM_CORES = 2
NUM_SUBCORES = 16
NUM_WORKERS = NUM_CORES * NUM_SUBCORES
CHUNK = 80
NCHUNK = 128
EDGES_PER_W = NCHUNK * CHUNK
N_PAD = 10240
TRASH_ROW = N_NODES
ROWS_PER_SUB = N_PAD // NUM_SUBCORES
VECS = CHUNK // 16


def _sc_aggregate(x, packed):
    mesh = plsc.VectorSubcoreMesh(core_axis_name="c", subcore_axis_name="s")

    @functools.partial(
        pl.kernel,
        mesh=mesh,
        out_type=jax.ShapeDtypeStruct((NUM_CORES, N_PAD, DIM), jnp.float32),
        scratch_types=[
            pltpu.VMEM((NCHUNK, CHUNK), jnp.int32),
            pltpu.VMEM((CHUNK,), jnp.int32),
            pltpu.VMEM((CHUNK,), jnp.int32),
            pltpu.VMEM((CHUNK,), jnp.int32),
            pltpu.VMEM((CHUNK,), jnp.int32),
            pltpu.VMEM((CHUNK, DIM), jnp.float32),
            pltpu.VMEM((CHUNK, DIM), jnp.float32),
            pltpu.VMEM_SHARED((N_PAD, DIM), jnp.float32),
            pltpu.SemaphoreType.DMA,
            pltpu.SemaphoreType.DMA,
        ],
    )
    def sc_kernel(x_hbm, pk_hbm, out_hbm,
                  pk_v, src0, src1, dst0, dst1, rows0, rows1,
                  aggr_sh, sem0, sem1):
        c = lax.axis_index("c")
        s = lax.axis_index("s")
        wid = c * NUM_SUBCORES + s

        idx_cp = pltpu.async_copy(pk_hbm.at[wid], pk_v, sem0)

        zeros16 = jnp.zeros((16,), jnp.float32)

        def zbody(i, carry):
            rows0[i // 8, pl.ds((i % 8) * 16, 16)] = zeros16
            return carry

        lax.fori_loop(0, CHUNK * 8, zbody, 0, unroll=8)

        base = s * ROWS_PER_SUB
        for r in range(ROWS_PER_SUB // CHUNK):
            pltpu.sync_copy(rows0,
                            aggr_sh.at[pl.ds(base + r * CHUNK, CHUNK)])
        idx_cp.wait()
        plsc.subcore_barrier()

        def unpack(j, src_b, dst_b):
            for t in range(VECS):
                w = pk_v[j, pl.ds(t * 16, 16)]
                src_b[pl.ds(t * 16, 16)] = lax.bitwise_and(w, 0xFFFF)
                dst_b[pl.ds(t * 16, 16)] = lax.shift_right_logical(w, 16)

        def gath(src_b, buf, sem):
            return pltpu.make_async_copy(x_hbm.at[src_b], buf, sem)

        unpack(0, src0, dst0)
        gath(src0, rows0, sem0).start()

        def body(i, carry):
            j = 2 * i
            unpack(j + 1, src1, dst1)
            gath(src1, rows1, sem1).start()
            gath(src0, rows0, sem0).wait()
            pltpu.sync_copy(rows0, aggr_sh.at[dst0], add=True)
            unpack(j + 2, src0, dst0)
            gath(src0, rows0, sem0).start()
            gath(src1, rows1, sem1).wait()
            pltpu.sync_copy(rows1, aggr_sh.at[dst1], add=True)
            return carry

        lax.fori_loop(0, NCHUNK // 2 - 1, body, 0)

        unpack(NCHUNK - 1, src1, dst1)
        gath(src1, rows1, sem1).start()
        gath(src0, rows0, sem0).wait()
        pltpu.sync_copy(rows0, aggr_sh.at[dst0], add=True)
        gath(src1, rows1, sem1).wait()
        pltpu.sync_copy(rows1, aggr_sh.at[dst1], add=True)
        plsc.subcore_barrier()

        pltpu.sync_copy(
            aggr_sh.at[pl.ds(base, ROWS_PER_SUB)],
            out_hbm.at[c, pl.ds(base, ROWS_PER_SUB)],
        )

    return sc_kernel(x, packed)


def _tc_tail(partials, x, W_l, b_l, W_r):
    def tc_kernel(p_ref, x_ref, wl_ref, wr_ref, bl_ref, o_ref):
        aggr = p_ref[0, :N_NODES, :] + p_ref[1, :N_NODES, :]
        h = lax.dot_general(
            aggr, wl_ref[...], (((1,), (1,)), ((), ())),
            preferred_element_type=jnp.float32,
        )
        h = h + lax.dot_general(
            x_ref[...], wr_ref[...], (((1,), (1,)), ((), ())),
            preferred_element_type=jnp.float32,
        )
        o_ref[...] = jnp.maximum(h + bl_ref[...], 0.0)

    return pl.pallas_call(
        tc_kernel,
        out_shape=jax.ShapeDtypeStruct((N_NODES, DIM), jnp.float32),
    )(partials, x, W_l, W_r, b_l.reshape(1, DIM))


@jax.jit
def kernel(x, edge_index, W_l, b_l, W_r):
    pad_w = EDGES_PER_W - E_EDGES // NUM_WORKERS
    src_p = jnp.concatenate(
        [edge_index[0].reshape(NUM_WORKERS, -1),
         jnp.zeros((NUM_WORKERS, pad_w), jnp.int32)], axis=1)
    dst_p = jnp.concatenate(
        [edge_index[1].reshape(NUM_WORKERS, -1),
         jnp.full((NUM_WORKERS, pad_w), TRASH_ROW, jnp.int32)], axis=1)
    packed = (jnp.left_shift(dst_p, 16) | src_p).reshape(
        NUM_WORKERS, NCHUNK, CHUNK)
    partials = _sc_aggregate(x, packed)
    return _tc_tail(partials, x, W_l, b_l, W_r)


# speedup vs baseline: 3.1492x; 3.1492x over previous
import functools
import jax
import jax.numpy as jnp
from jax import lax
from jax.experimental import pallas as pl
from jax.experimental.pallas import tpu as pltpu
from jax.experimental.pallas import tpu_sc as plsc

N_NODES = 10000
E_EDGES = 320000
DIM = 128

NUM_CORES = 2
NUM_SUBCORES = 16
NUM_WORKERS = NUM_CORES * NUM_SUBCORES
EDGES_PER_W = E_EDGES // NUM_WORKERS
CHUNK = 80
NCHUNK = EDGES_PER_W // CHUNK
STAGE = 10112
N_PAD = 10240
ROWS_PER_SUB = N_PAD // NUM_SUBCORES


def _sc_aggregate(x, edge_index):
    mesh = plsc.VectorSubcoreMesh(core_axis_name="c", subcore_axis_name="s")

    @functools.partial(
        pl.kernel,
        mesh=mesh,
        out_type=jax.ShapeDtypeStruct((NUM_CORES, N_PAD, DIM), jnp.float32),
        scratch_types=[
            pltpu.VMEM((STAGE,), jnp.int32),
            pltpu.VMEM((STAGE,), jnp.int32),
            pltpu.VMEM((CHUNK, DIM), jnp.float32),
            pltpu.VMEM((CHUNK, DIM), jnp.float32),
            pltpu.VMEM_SHARED((N_PAD, DIM), jnp.float32),
            pltpu.SemaphoreType.DMA,
            pltpu.SemaphoreType.DMA,
        ],
    )
    def sc_kernel(x_hbm, ei_hbm, out_hbm,
                  src_v, dst_v, rows0, rows1, aggr_sh, sem0, sem1):
        c = lax.axis_index("c")
        s = lax.axis_index("s")
        wid = c * NUM_SUBCORES + s

        off = pl.multiple_of((wid % 8) * 16, 16)
        lo = pl.multiple_of(wid * EDGES_PER_W - off, 128)
        idx_cp0 = pltpu.async_copy(
            ei_hbm.at[0, pl.ds(lo, STAGE)], src_v, sem0)
        idx_cp1 = pltpu.async_copy(
            ei_hbm.at[1, pl.ds(lo, STAGE)], dst_v, sem1)

        zeros16 = jnp.zeros((16,), jnp.float32)

        def zbody(i, carry):
            rows0[i // 8, pl.ds((i % 8) * 16, 16)] = zeros16
            return carry

        lax.fori_loop(0, CHUNK * 8, zbody, 0, unroll=8)

        base = s * ROWS_PER_SUB
        for r in range(ROWS_PER_SUB // CHUNK):
            pltpu.sync_copy(rows0,
                            aggr_sh.at[pl.ds(base + r * CHUNK, CHUNK)])
        idx_cp0.wait()
        idx_cp1.wait()
        plsc.subcore_barrier()

        def gref(j):
            return x_hbm.at[src_v.at[pl.ds(off + j * CHUNK, CHUNK)]]

        def dref(j):
            return aggr_sh.at[dst_v.at[pl.ds(off + j * CHUNK, CHUNK)]]

        pltpu.async_copy(gref(0), rows0, sem0)

        def body(i, carry):
            j = 2 * i
            pltpu.async_copy(gref(j + 1), rows1, sem1)
            pltpu.make_async_copy(gref(j), rows0, sem0).wait()
            pltpu.sync_copy(rows0, dref(j), add=True)
            pltpu.async_copy(gref(j + 2), rows0, sem0)
            pltpu.make_async_copy(gref(j + 1), rows1, sem1).wait()
            pltpu.sync_copy(rows1, dref(j + 1), add=True)
            return carry

        lax.fori_loop(0, (NCHUNK - 1) // 2, body, 0)

        pltpu.make_async_copy(gref(NCHUNK - 1), rows0, sem0).wait()
        pltpu.sync_copy(rows0, dref(NCHUNK - 1), add=True)
        plsc.subcore_barrier()

        pltpu.sync_copy(
            aggr_sh.at[pl.ds(base, ROWS_PER_SUB)],
            out_hbm.at[c, pl.ds(base, ROWS_PER_SUB)],
        )

    return sc_kernel(x, edge_index)


def _tc_tail(partials, x, W_l, b_l, W_r):
    def tc_kernel(p_ref, x_ref, wl_ref, wr_ref, bl_ref, o_ref):
        aggr = p_ref[0, :N_NODES, :] + p_ref[1, :N_NODES, :]
        h = lax.dot_general(
            aggr, wl_ref[...], (((1,), (1,)), ((), ())),
            preferred_element_type=jnp.float32,
        )
        h = h + lax.dot_general(
            x_ref[...], wr_ref[...], (((1,), (1,)), ((), ())),
            preferred_element_type=jnp.float32,
        )
        o_ref[...] = jnp.maximum(h + bl_ref[...], 0.0)

    return pl.pallas_call(
        tc_kernel,
        out_shape=jax.ShapeDtypeStruct((N_NODES, DIM), jnp.float32),
    )(partials, x, W_l, W_r, b_l.reshape(1, DIM))


@jax.jit
def kernel(x, edge_index, W_l, b_l, W_r):
    partials = _sc_aggregate(x, edge_index)
    return _tc_tail(partials, x, W_l, b_l, W_r)
